# Initial kernel scaffold; baseline (speedup 1.0000x reference)
import jax
import jax.numpy as jnp
from jax.experimental import pallas as pl


def kernel(x, conv0, norm0_s, norm0_b, block0_s1, block0_b1, block0_w1, block0_s2, block0_b2, block0_w2, block1_s1, block1_b1, block1_w1, block1_s2, block1_b2, block1_w2, block2_s1, block2_b1, block2_w1, block2_s2, block2_b2, block2_w2, block3_s1, block3_b1, block3_w1, block3_s2, block3_b2, block3_w2, trans0_scale2, trans0_shift2, trans0_conv, trans1_scale2, trans1_shift2, trans1_conv, trans2_scale2, trans2_shift2, trans2_conv, norm5_s, norm5_b, last_s, last_b, head_w, head_b):
    raise NotImplementedError("write your pallas kernel here")


# speedup vs baseline: 1.2166x; 1.2166x over previous
import jax
import jax.numpy as jnp
from jax.experimental import pallas as pl
from jax.experimental.pallas import tpu as pltpu

_RGB_MEAN = jnp.array([0.485, 0.456, 0.406], jnp.float32)
_RGB_STD = jnp.array([0.229, 0.224, 0.225], jnp.float32)
_GROWTH = 32
_BNECK = 128
_LAYERS = (6, 12, 24, 16)
_NCLS = (168, 11, 7)
_VMEM = 60 * 1024 * 1024


def _cp(*sem):
    return pltpu.CompilerParams(dimension_semantics=sem, vmem_limit_bytes=_VMEM)


def _stem_body(x_ref, w_ref, s_ref, b_ref, o_ref):
    acc = jnp.dot(x_ref[...], w_ref[...], preferred_element_type=jnp.float32)
    o_ref[...] = jnp.maximum(acc * s_ref[...] + b_ref[...], 0.0).astype(o_ref.dtype)


def _stem_matmul(cols2d, w, scale, shift):
    m, k = cols2d.shape
    n = w.shape[1]
    tm = 8192
    pm = ((m + tm - 1) // tm) * tm
    xp = cols2d if pm == m else jnp.pad(cols2d, ((0, pm - m), (0, 0)))
    out = pl.pallas_call(
        _stem_body,
        out_shape=jax.ShapeDtypeStruct((pm, n), jnp.bfloat16),
        grid=(pm // tm,),
        in_specs=[pl.BlockSpec((tm, k), lambda i: (i, 0)),
                  pl.BlockSpec((k, n), lambda i: (0, 0)),
                  pl.BlockSpec((1, n), lambda i: (0, 0)),
                  pl.BlockSpec((1, n), lambda i: (0, 0))],
        out_specs=pl.BlockSpec((tm, n), lambda i: (i, 0)),
        compiler_params=_cp("parallel"),
    )(xp, w, scale.reshape(1, n), shift.reshape(1, n))
    return out if pm == m else out[:m]


def _patches_7x7_s2(x):
    n, h, w, c = x.shape
    xp = jnp.pad(x, ((0, 0), (3, 3), (3, 3), (0, 0)))
    ho, wo = h // 2, w // 2
    taps = [xp[:, i:i + 2 * ho:2, j:j + 2 * wo:2, :]
            for i in range(7) for j in range(7)]
    return jnp.concatenate(taps, axis=-1), ho, wo


def _make_block(B, H, W, c_in, L):
    HW = H * W
    ctot = c_in + L * _GROWTH
    pad = W + 1
    hwpad = HW + 2 * pad

    def _body(x_ref, ml_ref, mr_ref, s1_ref, b1_ref, w1_ref,
              s2_ref, b2_ref, w2t_ref, o_ref, gbuf):
        o_ref[0, :, :c_in] = x_ref[0]
        gbuf[:pad, :] = jnp.zeros((pad, 9 * _GROWTH), jnp.float32)
        gbuf[pad + HW:, :] = jnp.zeros((pad, 9 * _GROWTH), jnp.float32)
        ml = ml_ref[...]
        mr = mr_ref[...]
        for i in range(L):
            cin = c_in + i * _GROWTH
            feat = o_ref[0, :, :cin].astype(jnp.float32)
            a = jnp.maximum(feat * s1_ref[i, :, :cin] + b1_ref[i, :, :cin], 0.0)
            h = jnp.dot(a.astype(jnp.bfloat16), w1_ref[i, :cin, :],
                        preferred_element_type=jnp.float32)
            h = jnp.maximum(h * s2_ref[i] + b2_ref[i], 0.0).astype(jnp.bfloat16)
            gbuf[pad:pad + HW, :] = jnp.dot(h, w2t_ref[i],
                                            preferred_element_type=jnp.float32)
            acc = jnp.zeros((HW, _GROWTH), jnp.float32)
            for dy in range(3):
                for dx in range(3):
                    t = dy * 3 + dx
                    off = dy * W + dx
                    tap = gbuf[off:off + HW, t * _GROWTH:(t + 1) * _GROWTH]
                    if dx == 0:
                        tap = tap * ml
                    elif dx == 2:
                        tap = tap * mr
                    acc = acc + tap
            o_ref[0, :, cin:cin + _GROWTH] = acc.astype(jnp.bfloat16)

    def rep(shape):
        return pl.BlockSpec(shape, lambda b: (0,) * len(shape))

    return pl.pallas_call(
        _body,
        out_shape=jax.ShapeDtypeStruct((B, HW, ctot), jnp.bfloat16),
        grid=(B,),
        in_specs=[
            pl.BlockSpec((1, HW, c_in), lambda b: (b, 0, 0)),
            rep((HW, 1)),
            rep((HW, 1)),
            rep((L, 1, ctot)),
            rep((L, 1, ctot)),
            rep((L, ctot, _BNECK)),
            rep((L, 1, _BNECK)),
            rep((L, 1, _BNECK)),
            rep((L, _BNECK, 9 * _GROWTH)),
        ],
        out_specs=pl.BlockSpec((1, HW, ctot), lambda b: (b, 0, 0)),
        scratch_shapes=[pltpu.VMEM((hwpad, 9 * _GROWTH), jnp.float32)],
        compiler_params=_cp("parallel"),
    )


def _make_transition(B, H, W, C):
    H2, W2, C2 = H // 2, W // 2, C // 2

    def _body(x_ref, s_ref, b_ref, w_ref, o_ref):
        x = x_ref[0].astype(jnp.float32)
        a = jnp.maximum(x * s_ref[...] + b_ref[...], 0.0)
        rs = a.reshape(H2, 2, W2, 2 * C).sum(axis=1)
        pooled = (rs[..., :C] + rs[..., C:]) * 0.25
        y = jnp.dot(pooled.reshape(H2 * W2, C).astype(jnp.bfloat16), w_ref[...],
                    preferred_element_type=jnp.float32)
        o_ref[0] = y.astype(jnp.bfloat16)

    return pl.pallas_call(
        _body,
        out_shape=jax.ShapeDtypeStruct((B, H2 * W2, C2), jnp.bfloat16),
        grid=(B,),
        in_specs=[
            pl.BlockSpec((1, H, W2, 2 * C), lambda b: (b, 0, 0, 0)),
            pl.BlockSpec((1, 1, 2 * C), lambda b: (0, 0, 0)),
            pl.BlockSpec((1, 1, 2 * C), lambda b: (0, 0, 0)),
            pl.BlockSpec((C, C2), lambda b: (0, 0)),
        ],
        out_specs=pl.BlockSpec((1, H2 * W2, C2), lambda b: (b, 0, 0)),
        compiler_params=_cp("parallel"),
    )


def _make_head(B, HW, C, n_out):
    inv = 1.0 / HW

    def _body(x_ref, s5_ref, b5_ref, sl_ref, bl_ref, w_ref, bias_ref, o_ref):
        x = x_ref[...].astype(jnp.float32)
        y = jnp.maximum(x * s5_ref[...] + b5_ref[...], 0.0)
        y = jnp.maximum(y * sl_ref[...] + bl_ref[...], 0.0)
        pooled = jnp.sum(y, axis=1) * inv
        o_ref[...] = (jnp.dot(pooled.astype(jnp.bfloat16), w_ref[...],
                              preferred_element_type=jnp.float32)
                      + bias_ref[...])

    return pl.pallas_call(
        _body,
        out_shape=jax.ShapeDtypeStruct((B, n_out), jnp.float32),
        grid=(1,),
        in_specs=[
            pl.BlockSpec((B, HW, C), lambda i: (0, 0, 0)),
            pl.BlockSpec((1, 1, C), lambda i: (0, 0, 0)),
            pl.BlockSpec((1, 1, C), lambda i: (0, 0, 0)),
            pl.BlockSpec((1, 1, C), lambda i: (0, 0, 0)),
            pl.BlockSpec((1, 1, C), lambda i: (0, 0, 0)),
            pl.BlockSpec((C, n_out), lambda i: (0, 0)),
            pl.BlockSpec((1, n_out), lambda i: (0, 0)),
        ],
        out_specs=pl.BlockSpec((B, n_out), lambda i: (0, 0)),
        compiler_params=_cp("arbitrary"),
    )


def kernel(x, conv0, norm0_s, norm0_b, block0_s1, block0_b1, block0_w1, block0_s2, block0_b2, block0_w2, block1_s1, block1_b1, block1_w1, block1_s2, block1_b2, block1_w2, block2_s1, block2_b1, block2_w1, block2_s2, block2_b2, block2_w2, block3_s1, block3_b1, block3_w1, block3_s2, block3_b2, block3_w2, trans0_scale2, trans0_shift2, trans0_conv, trans1_scale2, trans1_shift2, trans1_conv, trans2_scale2, trans2_shift2, trans2_conv, norm5_s, norm5_b, last_s, last_b, head_w, head_b):
    B = x.shape[0]
    xn = jnp.transpose(x, (0, 2, 3, 1)).astype(jnp.float32)
    xn = ((xn - _RGB_MEAN) / _RGB_STD).astype(jnp.bfloat16)

    cols, ho, wo = _patches_7x7_s2(xn)
    y = _stem_matmul(cols.reshape(B * ho * wo, cols.shape[-1]),
                     conv0, norm0_s, norm0_b)
    y = jax.lax.reduce_window(
        y.reshape(B, ho, wo, -1), jnp.array(-jnp.inf, y.dtype), jax.lax.max,
        window_dimensions=(1, 3, 3, 1), window_strides=(1, 2, 2, 1),
        padding=((0, 0), (1, 1), (1, 1), (0, 0)))
    _, h, w, c = y.shape
    feat = y.reshape(B, h * w, c)

    blocks = [
        (block0_s1, block0_b1, block0_w1, block0_s2, block0_b2, block0_w2),
        (block1_s1, block1_b1, block1_w1, block1_s2, block1_b2, block1_w2),
        (block2_s1, block2_b1, block2_w1, block2_s2, block2_b2, block2_w2),
        (block3_s1, block3_b1, block3_w1, block3_s2, block3_b2, block3_w2),
    ]
    trans = [
        (trans0_scale2, trans0_shift2, trans0_conv),
        (trans1_scale2, trans1_shift2, trans1_conv),
        (trans2_scale2, trans2_shift2, trans2_conv),
    ]
    for bi, L in enumerate(_LAYERS):
        s1, b1, w1, s2, b2, w2 = blocks[bi]
        hw = h * w
        widx = jnp.arange(hw, dtype=jnp.int32) % w
        ml = (widx != 0).astype(jnp.float32).reshape(hw, 1)
        mr = (widx != (w - 1)).astype(jnp.float32).reshape(hw, 1)
        w2t = jnp.transpose(w2, (0, 2, 1, 3)).reshape(L, _BNECK, 9 * _GROWTH)
        feat = _make_block(B, h, w, c, L)(feat, ml, mr, s1, b1, w1, s2, b2, w2t)
        c = c + L * _GROWTH
        if bi != len(_LAYERS) - 1:
            ts, tb, tw = trans[bi]
            xt = feat.reshape(B, h, w // 2, 2 * c)
            feat = _make_transition(B, h, w, c)(xt, ts, tb, tw)
            h, w, c = h // 2, w // 2, c // 2

    n_out = head_w.shape[1]
    logits = _make_head(B, h * w, c, n_out)(
        feat, norm5_s.reshape(1, 1, c), norm5_b.reshape(1, 1, c),
        last_s.reshape(1, 1, c), last_b.reshape(1, 1, c), head_w, head_b)
    n0, n1, n2 = _NCLS
    return [logits[:, :n0], logits[:, n0:n0 + n1], logits[:, n0 + n1:]]
